# baseline (speedup 1.0000x reference)
import jax
import jax.numpy as jnp
from jax import lax
from jax.experimental import pallas as pl
from jax.experimental.pallas import tpu as pltpu
from jax.experimental.pallas import tpu_sc as plsc

N = 10000
E = 320000
D = 128
NG = 64

NC = 2
NS = 16
NT = NC * NS
NPAD = 10240
EPT = 2 * E // NT
CSTREAM = 32
NRING = 8
EPTP = 20480
NROWP = EPTP // CSTREAM
NROWF = 624
TAIL = EPT - NROWF * CSTREAM
RBLK = 104
NBLK = NROWF // RBLK
NQUAD = RBLK // NRING
NACC = N
ROWS_T = N // NS
DEGC = NPAD // NS

_mesh = plsc.VectorSubcoreMesh(
    core_axis_name="c", subcore_axis_name="s", num_cores=NC, num_subcores=NS
)
_sc_params = pltpu.CompilerParams(
    needs_layout_passes=False, use_tc_tiling_on_sc=False
)


def _sc_deg_body(src_hbm, dst_hbm, deg_out, srcpad_out, dstpad_out,
                 hist_v, adj_v, dstf_v, red_v, out_v, deg_sh):
    c = lax.axis_index("c")
    s = lax.axis_index("s")
    tid = c * NS + s
    ebase = c * E + s * EPT

    def zero_body(j, _):
        hist_v[pl.ds(j * 16, 16)] = jnp.zeros((16,), jnp.float32)
        return 0
    lax.fori_loop(0, NPAD // 16, zero_body, 0)

    pltpu.sync_copy(src_hbm.at[pl.ds(ebase, EPT)], adj_v.at[pl.ds(0, EPT)])
    pltpu.sync_copy(dst_hbm.at[pl.ds(ebase, EPT)], dstf_v.at[pl.ds(0, EPT)])

    ones16 = jnp.full((16,), 1.0, jnp.float32)
    src_off = (c * N).astype(jnp.int32)

    def edge_body(j, _):
        d = dstf_v[pl.ds(j * 16, 16)]
        plsc.addupdate_scatter(hist_v, [d], ones16)
        adj_v[pl.ds(j * 16, 16)] = adj_v[pl.ds(j * 16, 16)] + src_off
        return 0
    lax.fori_loop(0, EPT // 16, edge_body, 0)

    def pad_body(j, _):
        adj_v[pl.ds(EPT + j * 16, 16)] = jnp.zeros((16,), jnp.int32)
        dstf_v[pl.ds(EPT + j * 16, 16)] = jnp.full((16,), N, jnp.int32)
        return 0
    lax.fori_loop(0, (EPTP - EPT) // 16, pad_body, 0)

    pltpu.sync_copy(adj_v, srcpad_out.at[pl.ds(tid * EPTP, EPTP)])
    pltpu.sync_copy(dstf_v, dstpad_out.at[pl.ds(tid * EPTP, EPTP)])

    pltpu.sync_copy(hist_v, deg_sh.at[s])
    plsc.subcore_barrier()
    for r in range(NS):
        pltpu.sync_copy(deg_sh.at[r, pl.ds(s * DEGC, DEGC)], red_v.at[r])

    def red_body(j, _):
        a = red_v[0, pl.ds(j * 16, 16)]
        for r in range(1, NS):
            a = a + red_v[r, pl.ds(j * 16, 16)]
        out_v[pl.ds(j * 16, 16)] = a
        return 0
    lax.fori_loop(0, DEGC // 16, red_body, 0)
    pltpu.sync_copy(out_v, deg_out.at[pl.ds(c * NPAD + s * DEGC, DEGC)])


_sc_deg = pl.kernel(
    _sc_deg_body,
    out_type=[
        jax.ShapeDtypeStruct((2 * NPAD,), jnp.float32),
        jax.ShapeDtypeStruct((NT * EPTP,), jnp.int32),
        jax.ShapeDtypeStruct((NT * EPTP,), jnp.int32),
    ],
    mesh=_mesh,
    scratch_types=[
        pltpu.VMEM((NPAD,), jnp.float32),
        pltpu.VMEM((EPTP,), jnp.int32),
        pltpu.VMEM((EPTP,), jnp.int32),
        pltpu.VMEM((NS, DEGC), jnp.float32),
        pltpu.VMEM((DEGC,), jnp.float32),
        pltpu.VMEM_SHARED((NS, NPAD), jnp.float32),
    ],
    compiler_params=_sc_params,
)


def _sc_conv_body(t_hbm, src2d_hbm, dst2d_hbm,
                  out_hbm, src_v, dst_v, src32, dst32,
                  rows0, rows1, rows2, rows3,
                  rows4, rows5, rows6, rows7, acc_sh, gsem):
    c = lax.axis_index("c")
    s = lax.axis_index("s")
    tid = c * NS + s
    ring = (rows0, rows1, rows2, rows3, rows4, rows5, rows6, rows7)

    pltpu.sync_copy(t_hbm.at[pl.ds(c * N + s * ROWS_T, ROWS_T)],
                    acc_sh.at[pl.ds(s * ROWS_T, ROWS_T)])
    plsc.subcore_barrier()

    rbase = tid * NROWP

    def blk_body(blk, _):
        pltpu.sync_copy(src2d_hbm.at[pl.ds(rbase + blk * RBLK, RBLK)], src_v)
        pltpu.sync_copy(dst2d_hbm.at[pl.ds(rbase + blk * RBLK, RBLK)], dst_v)
        for b in range(NRING):
            pltpu.async_copy(t_hbm.at[src_v.at[b]], ring[b], gsem)

        def ring_body(k, _):
            for b in range(NRING):
                j = NRING * k + b
                pltpu.make_async_copy(t_hbm.at[src_v.at[j]],
                                      ring[b], gsem).wait()
                pltpu.sync_copy(ring[b], acc_sh.at[dst_v.at[j]], add=True)

                @pl.when(k < NQUAD - 1)
                def _():
                    pltpu.async_copy(t_hbm.at[src_v.at[j + NRING]],
                                     ring[b], gsem)

            return 0
        lax.fori_loop(0, NQUAD, ring_body, 0)
        return 0
    lax.fori_loop(0, NBLK, blk_body, 0)

    trow = tid * NROWP + NROWF
    pltpu.sync_copy(src2d_hbm.at[trow, pl.ds(0, TAIL)], src32)
    pltpu.sync_copy(dst2d_hbm.at[trow, pl.ds(0, TAIL)], dst32)
    pltpu.async_copy(t_hbm.at[src32], rows0.at[pl.ds(0, TAIL)], gsem).wait()
    pltpu.sync_copy(rows0.at[pl.ds(0, TAIL)], acc_sh.at[dst32], add=True)

    plsc.subcore_barrier()
    pltpu.sync_copy(acc_sh.at[pl.ds(s * ROWS_T, ROWS_T)],
                    out_hbm.at[pl.ds(c * N + s * ROWS_T, ROWS_T)])


_sc_conv = pl.kernel(
    _sc_conv_body,
    out_type=jax.ShapeDtypeStruct((2 * N, D), jnp.float32),
    mesh=_mesh,
    scratch_types=[
        pltpu.VMEM((RBLK, CSTREAM), jnp.int32),
        pltpu.VMEM((RBLK, CSTREAM), jnp.int32),
        pltpu.VMEM((TAIL,), jnp.int32),
        pltpu.VMEM((TAIL,), jnp.int32),
        pltpu.VMEM((CSTREAM, D), jnp.float32),
        pltpu.VMEM((CSTREAM, D), jnp.float32),
        pltpu.VMEM((CSTREAM, D), jnp.float32),
        pltpu.VMEM((CSTREAM, D), jnp.float32),
        pltpu.VMEM((CSTREAM, D), jnp.float32),
        pltpu.VMEM((CSTREAM, D), jnp.float32),
        pltpu.VMEM((CSTREAM, D), jnp.float32),
        pltpu.VMEM((CSTREAM, D), jnp.float32),
        pltpu.VMEM_SHARED((NACC, D), jnp.float32),
        pltpu.SemaphoreType.DMA,
    ],
    compiler_params=_sc_params,
)


RB = 400
NRB = N // RB


def _dinv(deg_blk):
    return 1.0 / jnp.sqrt(deg_blk + 1.0)


def _tc_t0_body(x_ref, deg_ref, w_ref, o_ref):
    z = jnp.dot(x_ref[0], w_ref[...], preferred_element_type=jnp.float32)
    o_ref[0] = z * _dinv(deg_ref[0])


def _tc_layer_body(s_ref, deg_ref, b_ref, w_ref, o_ref):
    dinv = _dinv(deg_ref[0])
    h = jnp.maximum(dinv * s_ref[0] + b_ref[...], 0.0)
    o_ref[0] = jnp.dot(h, w_ref[...], preferred_element_type=jnp.float32) * dinv


def _tc_pool_body(s_ref, deg_ref, b_ref, batch_ref,
                  w0_ref, b0_ref, w1_ref, b1_ref, w2_ref, b2_ref,
                  p_ref, o_ref):
    g = pl.program_id(0)
    i = pl.program_id(1)
    h = jnp.maximum(_dinv(deg_ref[0]) * s_ref[0] + b_ref[...], 0.0)
    bt = batch_ref[0, 0]
    oh = (bt[:, None] == lax.broadcasted_iota(jnp.int32, (RB, NG), 1))
    pp = lax.dot_general(oh.astype(jnp.float32), h,
                         (((0,), (0,)), ((), ())),
                         preferred_element_type=jnp.float32)

    @pl.when(jnp.logical_and(g == 0, i == 0))
    def _():
        p_ref[0] = pp

    @pl.when(jnp.logical_and(g == 0, i > 0))
    def _():
        p_ref[0] += pp

    @pl.when(jnp.logical_and(g == 1, i == 0))
    def _():
        p_ref[1] = pp

    @pl.when(jnp.logical_and(g == 1, i > 0))
    def _():
        p_ref[1] += pp

    @pl.when(jnp.logical_and(g == 1, i == NRB - 1))
    def _():
        z = (jnp.dot(p_ref[0], w0_ref[0:D, :],
                     preferred_element_type=jnp.float32)
             + jnp.dot(p_ref[1], w0_ref[D:2 * D, :],
                       preferred_element_type=jnp.float32) + b0_ref[...])
        a = jnp.maximum(z, 0.0)
        a = jnp.maximum(jnp.dot(a, w1_ref[...],
                                preferred_element_type=jnp.float32)
                        + b1_ref[...], 0.0)
        z2 = (jnp.dot(a, w2_ref[...], preferred_element_type=jnp.float32)
              + b2_ref[...])
        o_ref[...] = jax.nn.sigmoid(z2)


_tc_t0 = pl.pallas_call(
    _tc_t0_body,
    grid=(2, NRB),
    in_specs=[
        pl.BlockSpec((1, RB, D), lambda g, i: (g, i, 0)),
        pl.BlockSpec((1, RB, 1), lambda g, i: (g, i, 0)),
        pl.BlockSpec((D, D), lambda g, i: (0, 0)),
    ],
    out_specs=pl.BlockSpec((1, RB, D), lambda g, i: (g, i, 0)),
    out_shape=jax.ShapeDtypeStruct((2, N, D), jnp.float32),
)

_tc_layer = pl.pallas_call(
    _tc_layer_body,
    grid=(2, NRB),
    in_specs=[
        pl.BlockSpec((1, RB, D), lambda g, i: (g, i, 0)),
        pl.BlockSpec((1, RB, 1), lambda g, i: (g, i, 0)),
        pl.BlockSpec((1, D), lambda g, i: (0, 0)),
        pl.BlockSpec((D, D), lambda g, i: (0, 0)),
    ],
    out_specs=pl.BlockSpec((1, RB, D), lambda g, i: (g, i, 0)),
    out_shape=jax.ShapeDtypeStruct((2, N, D), jnp.float32),
)

_tc_pool = pl.pallas_call(
    _tc_pool_body,
    grid=(2, NRB),
    in_specs=[
        pl.BlockSpec((1, RB, D), lambda g, i: (g, i, 0)),
        pl.BlockSpec((1, RB, 1), lambda g, i: (g, i, 0)),
        pl.BlockSpec((1, D), lambda g, i: (0, 0)),
        pl.BlockSpec((1, 1, RB), lambda g, i: (g * NRB + i, 0, 0)),
        pl.BlockSpec((2 * D, D), lambda g, i: (0, 0)),
        pl.BlockSpec((1, D), lambda g, i: (0, 0)),
        pl.BlockSpec((D, D // 2), lambda g, i: (0, 0)),
        pl.BlockSpec((1, D // 2), lambda g, i: (0, 0)),
        pl.BlockSpec((D // 2, D), lambda g, i: (0, 0)),
        pl.BlockSpec((1, D), lambda g, i: (0, 0)),
    ],
    out_specs=[
        pl.BlockSpec((2, NG, D), lambda g, i: (0, 0, 0)),
        pl.BlockSpec((NG, D), lambda g, i: (0, 0)),
    ],
    out_shape=[
        jax.ShapeDtypeStruct((2, NG, D), jnp.float32),
        jax.ShapeDtypeStruct((NG, D), jnp.float32),
    ],
)


def kernel(x1, edge_index1, batch1, x2, edge_index2, batch2,
           Wg0, bg0, Wg1, bg1, Wg2, bg2, W0, b0, W1, b1, W2, b2):
    x_all = jnp.stack([x1, x2])
    src_cat = jnp.concatenate([edge_index1[0], edge_index2[0]]).astype(jnp.int32)
    dst_cat = jnp.concatenate([edge_index1[1], edge_index2[1]]).astype(jnp.int32)

    deg_flat, srcpad, dstpad = _sc_deg(src_cat, dst_cat)
    deg = deg_flat.reshape(2, NPAD)[:, :N].reshape(2, N, 1)
    src2d = srcpad.reshape(NT * EPTP // CSTREAM, CSTREAM)
    dst2d = dstpad.reshape(NT * EPTP // CSTREAM, CSTREAM)

    t = _tc_t0(x_all, deg, Wg0)
    for W_next, b_prev in ((Wg1, bg0), (Wg2, bg1)):
        s_ = _sc_conv(t.reshape(2 * N, D), src2d, dst2d).reshape(2, N, D)
        t = _tc_layer(s_, deg, b_prev.reshape(1, D), W_next)
    s_ = _sc_conv(t.reshape(2 * N, D), src2d, dst2d).reshape(2, N, D)

    batch3d = jnp.stack([batch1, batch2]).astype(jnp.int32).reshape(2 * NRB, 1, RB)
    W2p = jnp.pad(W2, ((0, 0), (0, D - 1)))
    b2p = jnp.pad(b2, (0, D - 1)).reshape(1, D)
    _, out = _tc_pool(s_, deg, bg2.reshape(1, D), batch3d,
                      W0, b0.reshape(1, D), W1, b1.reshape(1, D // 2),
                      W2p, b2p)
    return out[:, 0]


# speedup vs baseline: 1.0255x; 1.0255x over previous
import jax
import jax.numpy as jnp
from jax import lax
from jax.experimental import pallas as pl
from jax.experimental.pallas import tpu as pltpu
from jax.experimental.pallas import tpu_sc as plsc

N = 10000
E = 320000
D = 128
NG = 64

NC = 2
NS = 16
NT = NC * NS
NPAD = 10240
EPT = 2 * E // NT
CSTREAM = 64
NRING = 4
EPTP = 20480
NROWP = EPTP // CSTREAM
NROWF = EPT // CSTREAM
TAIL = EPT - NROWF * CSTREAM
RBLK = 52
NBLK = NROWF // RBLK
NQUAD = RBLK // NRING
NACC = N
ROWS_T = N // NS
DEGC = NPAD // NS

_mesh = plsc.VectorSubcoreMesh(
    core_axis_name="c", subcore_axis_name="s", num_cores=NC, num_subcores=NS
)
_sc_params = pltpu.CompilerParams(
    needs_layout_passes=False, use_tc_tiling_on_sc=False
)


def _sc_deg_body(src_hbm, dst_hbm, deg_out, srcpad_out, dstpad_out,
                 hist_v, adj_v, dstf_v, red_v, out_v, deg_sh):
    c = lax.axis_index("c")
    s = lax.axis_index("s")
    tid = c * NS + s
    ebase = c * E + s * EPT

    def zero_body(j, _):
        hist_v[pl.ds(j * 16, 16)] = jnp.zeros((16,), jnp.float32)
        return 0
    lax.fori_loop(0, NPAD // 16, zero_body, 0)

    pltpu.sync_copy(src_hbm.at[pl.ds(ebase, EPT)], adj_v.at[pl.ds(0, EPT)])
    pltpu.sync_copy(dst_hbm.at[pl.ds(ebase, EPT)], dstf_v.at[pl.ds(0, EPT)])

    ones16 = jnp.full((16,), 1.0, jnp.float32)
    src_off = (c * N).astype(jnp.int32)

    def edge_body(j, _):
        d = dstf_v[pl.ds(j * 16, 16)]
        plsc.addupdate_scatter(hist_v, [d], ones16)
        adj_v[pl.ds(j * 16, 16)] = adj_v[pl.ds(j * 16, 16)] + src_off
        return 0
    lax.fori_loop(0, EPT // 16, edge_body, 0)

    def pad_body(j, _):
        adj_v[pl.ds(EPT + j * 16, 16)] = jnp.zeros((16,), jnp.int32)
        dstf_v[pl.ds(EPT + j * 16, 16)] = jnp.full((16,), N, jnp.int32)
        return 0
    lax.fori_loop(0, (EPTP - EPT) // 16, pad_body, 0)

    pltpu.sync_copy(adj_v, srcpad_out.at[pl.ds(tid * EPTP, EPTP)])
    pltpu.sync_copy(dstf_v, dstpad_out.at[pl.ds(tid * EPTP, EPTP)])

    pltpu.sync_copy(hist_v, deg_sh.at[s])
    plsc.subcore_barrier()
    for r in range(NS):
        pltpu.sync_copy(deg_sh.at[r, pl.ds(s * DEGC, DEGC)], red_v.at[r])

    def red_body(j, _):
        a = red_v[0, pl.ds(j * 16, 16)]
        for r in range(1, NS):
            a = a + red_v[r, pl.ds(j * 16, 16)]
        out_v[pl.ds(j * 16, 16)] = a
        return 0
    lax.fori_loop(0, DEGC // 16, red_body, 0)
    pltpu.sync_copy(out_v, deg_out.at[pl.ds(c * NPAD + s * DEGC, DEGC)])


_sc_deg = pl.kernel(
    _sc_deg_body,
    out_type=[
        jax.ShapeDtypeStruct((2 * NPAD,), jnp.float32),
        jax.ShapeDtypeStruct((NT * EPTP,), jnp.int32),
        jax.ShapeDtypeStruct((NT * EPTP,), jnp.int32),
    ],
    mesh=_mesh,
    scratch_types=[
        pltpu.VMEM((NPAD,), jnp.float32),
        pltpu.VMEM((EPTP,), jnp.int32),
        pltpu.VMEM((EPTP,), jnp.int32),
        pltpu.VMEM((NS, DEGC), jnp.float32),
        pltpu.VMEM((DEGC,), jnp.float32),
        pltpu.VMEM_SHARED((NS, NPAD), jnp.float32),
    ],
    compiler_params=_sc_params,
)


def _sc_conv_body(t_hbm, src2d_hbm, dst2d_hbm,
                  out_hbm, src_a, dst_a, src_b, dst_b, src32, dst32,
                  rows0, rows1, rows2, rows3, acc_sh, gsem, isem):
    c = lax.axis_index("c")
    s = lax.axis_index("s")
    tid = c * NS + s
    ring = (rows0, rows1, rows2, rows3)
    ibufs = ((src_a, dst_a), (src_b, dst_b))

    pltpu.sync_copy(t_hbm.at[pl.ds(c * N + s * ROWS_T, ROWS_T)],
                    acc_sh.at[pl.ds(s * ROWS_T, ROWS_T)])
    plsc.subcore_barrier()

    rbase = tid * NROWP

    pltpu.sync_copy(src2d_hbm.at[pl.ds(rbase, RBLK)], src_a)
    pltpu.sync_copy(dst2d_hbm.at[pl.ds(rbase, RBLK)], dst_a)
    for blk in range(NBLK):
        src_v, dst_v = ibufs[blk % 2]
        nsv, ndv = ibufs[(blk + 1) % 2]
        if blk + 1 < NBLK:
            r1 = rbase + (blk + 1) * RBLK
            pltpu.async_copy(src2d_hbm.at[pl.ds(r1, RBLK)], nsv, isem)
            pltpu.async_copy(dst2d_hbm.at[pl.ds(r1, RBLK)], ndv, isem)
        for b in range(NRING):
            pltpu.async_copy(t_hbm.at[src_v.at[b]], ring[b], gsem)

        def ring_body(k, _, src_v=src_v, dst_v=dst_v):
            for b in range(NRING):
                j = NRING * k + b
                pltpu.make_async_copy(t_hbm.at[src_v.at[j]],
                                      ring[b], gsem).wait()
                pltpu.sync_copy(ring[b], acc_sh.at[dst_v.at[j]], add=True)

                @pl.when(k < NQUAD - 1)
                def _():
                    pltpu.async_copy(t_hbm.at[src_v.at[j + NRING]],
                                     ring[b], gsem)

            return 0
        lax.fori_loop(0, NQUAD, ring_body, 0)
        if blk + 1 < NBLK:
            r1 = rbase + (blk + 1) * RBLK
            pltpu.make_async_copy(src2d_hbm.at[pl.ds(r1, RBLK)], nsv,
                                  isem).wait()
            pltpu.make_async_copy(dst2d_hbm.at[pl.ds(r1, RBLK)], ndv,
                                  isem).wait()

    trow = tid * NROWP + NROWF
    pltpu.sync_copy(src2d_hbm.at[trow, pl.ds(0, TAIL)], src32)
    pltpu.sync_copy(dst2d_hbm.at[trow, pl.ds(0, TAIL)], dst32)
    pltpu.async_copy(t_hbm.at[src32], rows0.at[pl.ds(0, TAIL)], gsem).wait()
    pltpu.sync_copy(rows0.at[pl.ds(0, TAIL)], acc_sh.at[dst32], add=True)

    plsc.subcore_barrier()
    pltpu.sync_copy(acc_sh.at[pl.ds(s * ROWS_T, ROWS_T)],
                    out_hbm.at[pl.ds(c * N + s * ROWS_T, ROWS_T)])


_sc_conv = pl.kernel(
    _sc_conv_body,
    out_type=jax.ShapeDtypeStruct((2 * N, D), jnp.float32),
    mesh=_mesh,
    scratch_types=[
        pltpu.VMEM((RBLK, CSTREAM), jnp.int32),
        pltpu.VMEM((RBLK, CSTREAM), jnp.int32),
        pltpu.VMEM((RBLK, CSTREAM), jnp.int32),
        pltpu.VMEM((RBLK, CSTREAM), jnp.int32),
        pltpu.VMEM((TAIL,), jnp.int32),
        pltpu.VMEM((TAIL,), jnp.int32),
        pltpu.VMEM((CSTREAM, D), jnp.float32),
        pltpu.VMEM((CSTREAM, D), jnp.float32),
        pltpu.VMEM((CSTREAM, D), jnp.float32),
        pltpu.VMEM((CSTREAM, D), jnp.float32),
        pltpu.VMEM_SHARED((NACC, D), jnp.float32),
        pltpu.SemaphoreType.DMA,
        pltpu.SemaphoreType.DMA,
    ],
    compiler_params=_sc_params,
)


RB = 400
NRB = N // RB


def _dinv(deg_blk):
    return 1.0 / jnp.sqrt(deg_blk + 1.0)


def _tc_t0_body(x_ref, deg_ref, w_ref, o_ref):
    z = jnp.dot(x_ref[0], w_ref[...], preferred_element_type=jnp.float32)
    o_ref[0] = z * _dinv(deg_ref[0])


def _tc_layer_body(s_ref, deg_ref, b_ref, w_ref, o_ref):
    dinv = _dinv(deg_ref[0])
    h = jnp.maximum(dinv * s_ref[0] + b_ref[...], 0.0)
    o_ref[0] = jnp.dot(h, w_ref[...], preferred_element_type=jnp.float32) * dinv


def _tc_pool_body(s_ref, deg_ref, b_ref, batch_ref,
                  w0_ref, b0_ref, w1_ref, b1_ref, w2_ref, b2_ref,
                  p_ref, o_ref):
    g = pl.program_id(0)
    i = pl.program_id(1)
    h = jnp.maximum(_dinv(deg_ref[0]) * s_ref[0] + b_ref[...], 0.0)
    bt = batch_ref[0, 0]
    oh = (bt[:, None] == lax.broadcasted_iota(jnp.int32, (RB, NG), 1))
    pp = lax.dot_general(oh.astype(jnp.float32), h,
                         (((0,), (0,)), ((), ())),
                         preferred_element_type=jnp.float32)

    @pl.when(jnp.logical_and(g == 0, i == 0))
    def _():
        p_ref[0] = pp

    @pl.when(jnp.logical_and(g == 0, i > 0))
    def _():
        p_ref[0] += pp

    @pl.when(jnp.logical_and(g == 1, i == 0))
    def _():
        p_ref[1] = pp

    @pl.when(jnp.logical_and(g == 1, i > 0))
    def _():
        p_ref[1] += pp

    @pl.when(jnp.logical_and(g == 1, i == NRB - 1))
    def _():
        z = (jnp.dot(p_ref[0], w0_ref[0:D, :],
                     preferred_element_type=jnp.float32)
             + jnp.dot(p_ref[1], w0_ref[D:2 * D, :],
                       preferred_element_type=jnp.float32) + b0_ref[...])
        a = jnp.maximum(z, 0.0)
        a = jnp.maximum(jnp.dot(a, w1_ref[...],
                                preferred_element_type=jnp.float32)
                        + b1_ref[...], 0.0)
        z2 = (jnp.dot(a, w2_ref[...], preferred_element_type=jnp.float32)
              + b2_ref[...])
        o_ref[...] = jax.nn.sigmoid(z2)


_tc_t0 = pl.pallas_call(
    _tc_t0_body,
    grid=(2, NRB),
    in_specs=[
        pl.BlockSpec((1, RB, D), lambda g, i: (g, i, 0)),
        pl.BlockSpec((1, RB, 1), lambda g, i: (g, i, 0)),
        pl.BlockSpec((D, D), lambda g, i: (0, 0)),
    ],
    out_specs=pl.BlockSpec((1, RB, D), lambda g, i: (g, i, 0)),
    out_shape=jax.ShapeDtypeStruct((2, N, D), jnp.float32),
)

_tc_layer = pl.pallas_call(
    _tc_layer_body,
    grid=(2, NRB),
    in_specs=[
        pl.BlockSpec((1, RB, D), lambda g, i: (g, i, 0)),
        pl.BlockSpec((1, RB, 1), lambda g, i: (g, i, 0)),
        pl.BlockSpec((1, D), lambda g, i: (0, 0)),
        pl.BlockSpec((D, D), lambda g, i: (0, 0)),
    ],
    out_specs=pl.BlockSpec((1, RB, D), lambda g, i: (g, i, 0)),
    out_shape=jax.ShapeDtypeStruct((2, N, D), jnp.float32),
)

_tc_pool = pl.pallas_call(
    _tc_pool_body,
    grid=(2, NRB),
    in_specs=[
        pl.BlockSpec((1, RB, D), lambda g, i: (g, i, 0)),
        pl.BlockSpec((1, RB, 1), lambda g, i: (g, i, 0)),
        pl.BlockSpec((1, D), lambda g, i: (0, 0)),
        pl.BlockSpec((1, 1, RB), lambda g, i: (g * NRB + i, 0, 0)),
        pl.BlockSpec((2 * D, D), lambda g, i: (0, 0)),
        pl.BlockSpec((1, D), lambda g, i: (0, 0)),
        pl.BlockSpec((D, D // 2), lambda g, i: (0, 0)),
        pl.BlockSpec((1, D // 2), lambda g, i: (0, 0)),
        pl.BlockSpec((D // 2, D), lambda g, i: (0, 0)),
        pl.BlockSpec((1, D), lambda g, i: (0, 0)),
    ],
    out_specs=[
        pl.BlockSpec((2, NG, D), lambda g, i: (0, 0, 0)),
        pl.BlockSpec((NG, D), lambda g, i: (0, 0)),
    ],
    out_shape=[
        jax.ShapeDtypeStruct((2, NG, D), jnp.float32),
        jax.ShapeDtypeStruct((NG, D), jnp.float32),
    ],
)


def kernel(x1, edge_index1, batch1, x2, edge_index2, batch2,
           Wg0, bg0, Wg1, bg1, Wg2, bg2, W0, b0, W1, b1, W2, b2):
    x_all = jnp.stack([x1, x2])
    src_cat = jnp.concatenate([edge_index1[0], edge_index2[0]]).astype(jnp.int32)
    dst_cat = jnp.concatenate([edge_index1[1], edge_index2[1]]).astype(jnp.int32)

    deg_flat, srcpad, dstpad = _sc_deg(src_cat, dst_cat)
    deg = deg_flat.reshape(2, NPAD)[:, :N].reshape(2, N, 1)
    src2d = srcpad.reshape(NT * EPTP // CSTREAM, CSTREAM)
    dst2d = dstpad.reshape(NT * EPTP // CSTREAM, CSTREAM)

    t = _tc_t0(x_all, deg, Wg0)
    for W_next, b_prev in ((Wg1, bg0), (Wg2, bg1)):
        s_ = _sc_conv(t.reshape(2 * N, D), src2d, dst2d).reshape(2, N, D)
        t = _tc_layer(s_, deg, b_prev.reshape(1, D), W_next)
    s_ = _sc_conv(t.reshape(2 * N, D), src2d, dst2d).reshape(2, N, D)

    batch3d = jnp.stack([batch1, batch2]).astype(jnp.int32).reshape(2 * NRB, 1, RB)
    W2p = jnp.pad(W2, ((0, 0), (0, D - 1)))
    b2p = jnp.pad(b2, (0, D - 1)).reshape(1, D)
    _, out = _tc_pool(s_, deg, bg2.reshape(1, D), batch3d,
                      W0, b0.reshape(1, D), W1, b1.reshape(1, D // 2),
                      W2p, b2p)
    return out[:, 0]


# speedup vs baseline: 1.0491x; 1.0230x over previous
import jax
import jax.numpy as jnp
from jax import lax
from jax.experimental import pallas as pl
from jax.experimental.pallas import tpu as pltpu
from jax.experimental.pallas import tpu_sc as plsc

N = 10000
E = 320000
D = 128
NG = 64

NC = 2
NS = 16
NT = NC * NS
NPAD = 10240
EPT = 2 * E // NT
CSTREAM = 64
NRING = 4
EPTP = 20480
NROWP = EPTP // CSTREAM
NROWF = EPT // CSTREAM
TAIL = EPT - NROWF * CSTREAM
RBLK = 52
NBLK = NROWF // RBLK
NQUAD = RBLK // NRING
NACC = N
ROWS_T = N // NS
DEGC = NPAD // NS

_mesh = plsc.VectorSubcoreMesh(
    core_axis_name="c", subcore_axis_name="s", num_cores=NC, num_subcores=NS
)
_sc_params = pltpu.CompilerParams(
    needs_layout_passes=False, use_tc_tiling_on_sc=False
)


def _sc_deg_body(src_hbm, dst_hbm, deg_out, srcpad_out, dstpad_out,
                 hist_v, adj_v, dstf_v, red_v, out_v, deg_sh):
    c = lax.axis_index("c")
    s = lax.axis_index("s")
    tid = c * NS + s
    ebase = c * E + s * EPT

    def zero_body(j, _):
        hist_v[pl.ds(j * 16, 16)] = jnp.zeros((16,), jnp.float32)
        return 0
    lax.fori_loop(0, NPAD // 16, zero_body, 0)

    pltpu.sync_copy(src_hbm.at[pl.ds(ebase, EPT)], adj_v.at[pl.ds(0, EPT)])
    pltpu.sync_copy(dst_hbm.at[pl.ds(ebase, EPT)], dstf_v.at[pl.ds(0, EPT)])

    ones16 = jnp.full((16,), 1.0, jnp.float32)
    src_off = (c * N).astype(jnp.int32)

    def edge_body(j, _):
        d = dstf_v[pl.ds(j * 16, 16)]
        plsc.addupdate_scatter(hist_v, [d], ones16)
        adj_v[pl.ds(j * 16, 16)] = adj_v[pl.ds(j * 16, 16)] + src_off
        return 0
    lax.fori_loop(0, EPT // 16, edge_body, 0)

    def pad_body(j, _):
        adj_v[pl.ds(EPT + j * 16, 16)] = jnp.zeros((16,), jnp.int32)
        dstf_v[pl.ds(EPT + j * 16, 16)] = jnp.full((16,), N, jnp.int32)
        return 0
    lax.fori_loop(0, (EPTP - EPT) // 16, pad_body, 0)

    pltpu.sync_copy(adj_v, srcpad_out.at[pl.ds(tid * EPTP, EPTP)])
    pltpu.sync_copy(dstf_v, dstpad_out.at[pl.ds(tid * EPTP, EPTP)])

    pltpu.sync_copy(hist_v, deg_sh.at[s])
    plsc.subcore_barrier()
    for r in range(NS):
        pltpu.sync_copy(deg_sh.at[r, pl.ds(s * DEGC, DEGC)], red_v.at[r])

    def red_body(j, _):
        a = red_v[0, pl.ds(j * 16, 16)]
        for r in range(1, NS):
            a = a + red_v[r, pl.ds(j * 16, 16)]
        out_v[pl.ds(j * 16, 16)] = a
        return 0
    lax.fori_loop(0, DEGC // 16, red_body, 0)
    pltpu.sync_copy(out_v, deg_out.at[pl.ds(c * NPAD + s * DEGC, DEGC)])


_sc_deg = pl.kernel(
    _sc_deg_body,
    out_type=[
        jax.ShapeDtypeStruct((2 * NPAD,), jnp.float32),
        jax.ShapeDtypeStruct((NT * EPTP,), jnp.int32),
        jax.ShapeDtypeStruct((NT * EPTP,), jnp.int32),
    ],
    mesh=_mesh,
    scratch_types=[
        pltpu.VMEM((NPAD,), jnp.float32),
        pltpu.VMEM((EPTP,), jnp.int32),
        pltpu.VMEM((EPTP,), jnp.int32),
        pltpu.VMEM((NS, DEGC), jnp.float32),
        pltpu.VMEM((DEGC,), jnp.float32),
        pltpu.VMEM_SHARED((NS, NPAD), jnp.float32),
    ],
    compiler_params=_sc_params,
)


def _sc_conv_body(t_hbm, src2d_hbm, dst2d_hbm,
                  out_hbm, src_a, dst_a, src_b, dst_b, src32, dst32,
                  rows0, rows1, rows2, rows3, acc_sh, gsem, isem):
    c = lax.axis_index("c")
    s = lax.axis_index("s")
    tid = c * NS + s
    ring = (rows0, rows1, rows2, rows3)
    ibufs = ((src_a, dst_a), (src_b, dst_b))

    pltpu.sync_copy(t_hbm.at[pl.ds(c * N + s * ROWS_T, ROWS_T)],
                    acc_sh.at[pl.ds(s * ROWS_T, ROWS_T)])
    plsc.subcore_barrier()

    rbase = tid * NROWP

    pltpu.sync_copy(src2d_hbm.at[pl.ds(rbase, RBLK)], src_a)
    pltpu.sync_copy(dst2d_hbm.at[pl.ds(rbase, RBLK)], dst_a)
    for b in range(NRING):
        pltpu.async_copy(t_hbm.at[src_a.at[b]], ring[b], gsem)

    for blk in range(NBLK):
        src_v, dst_v = ibufs[blk % 2]
        nsv, ndv = ibufs[(blk + 1) % 2]
        if blk + 1 < NBLK:
            r1 = rbase + (blk + 1) * RBLK
            pltpu.async_copy(src2d_hbm.at[pl.ds(r1, RBLK)], nsv, isem)
            pltpu.async_copy(dst2d_hbm.at[pl.ds(r1, RBLK)], ndv, isem)

        def ring_body(k, _, src_v=src_v, dst_v=dst_v):
            for b in range(NRING):
                j = NRING * k + b
                pltpu.make_async_copy(t_hbm.at[src_v.at[j]],
                                      ring[b], gsem).wait()
                pltpu.sync_copy(ring[b], acc_sh.at[dst_v.at[j]], add=True)
                pltpu.async_copy(t_hbm.at[src_v.at[j + NRING]],
                                 ring[b], gsem)
            return 0
        lax.fori_loop(0, NQUAD - 1, ring_body, 0)

        if blk + 1 < NBLK:
            r1 = rbase + (blk + 1) * RBLK
            pltpu.make_async_copy(src2d_hbm.at[pl.ds(r1, RBLK)], nsv,
                                  isem).wait()
            pltpu.make_async_copy(dst2d_hbm.at[pl.ds(r1, RBLK)], ndv,
                                  isem).wait()
        for b in range(NRING):
            j = NRING * (NQUAD - 1) + b
            pltpu.make_async_copy(t_hbm.at[src_v.at[j]],
                                  ring[b], gsem).wait()
            pltpu.sync_copy(ring[b], acc_sh.at[dst_v.at[j]], add=True)
            if blk + 1 < NBLK:
                pltpu.async_copy(t_hbm.at[nsv.at[b]], ring[b], gsem)

    trow = tid * NROWP + NROWF
    pltpu.sync_copy(src2d_hbm.at[trow, pl.ds(0, TAIL)], src32)
    pltpu.sync_copy(dst2d_hbm.at[trow, pl.ds(0, TAIL)], dst32)
    pltpu.async_copy(t_hbm.at[src32], rows0.at[pl.ds(0, TAIL)], gsem).wait()
    pltpu.sync_copy(rows0.at[pl.ds(0, TAIL)], acc_sh.at[dst32], add=True)

    plsc.subcore_barrier()
    pltpu.sync_copy(acc_sh.at[pl.ds(s * ROWS_T, ROWS_T)],
                    out_hbm.at[pl.ds(c * N + s * ROWS_T, ROWS_T)])


_sc_conv = pl.kernel(
    _sc_conv_body,
    out_type=jax.ShapeDtypeStruct((2 * N, D), jnp.float32),
    mesh=_mesh,
    scratch_types=[
        pltpu.VMEM((RBLK, CSTREAM), jnp.int32),
        pltpu.VMEM((RBLK, CSTREAM), jnp.int32),
        pltpu.VMEM((RBLK, CSTREAM), jnp.int32),
        pltpu.VMEM((RBLK, CSTREAM), jnp.int32),
        pltpu.VMEM((TAIL,), jnp.int32),
        pltpu.VMEM((TAIL,), jnp.int32),
        pltpu.VMEM((CSTREAM, D), jnp.float32),
        pltpu.VMEM((CSTREAM, D), jnp.float32),
        pltpu.VMEM((CSTREAM, D), jnp.float32),
        pltpu.VMEM((CSTREAM, D), jnp.float32),
        pltpu.VMEM_SHARED((NACC, D), jnp.float32),
        pltpu.SemaphoreType.DMA,
        pltpu.SemaphoreType.DMA,
    ],
    compiler_params=_sc_params,
)


RB = 400
NRB = N // RB


def _dinv(deg_blk):
    return 1.0 / jnp.sqrt(deg_blk + 1.0)


def _tc_t0_body(x_ref, deg_ref, w_ref, o_ref):
    z = jnp.dot(x_ref[0], w_ref[...], preferred_element_type=jnp.float32)
    o_ref[0] = z * _dinv(deg_ref[0])


def _tc_layer_body(s_ref, deg_ref, b_ref, w_ref, o_ref):
    dinv = _dinv(deg_ref[0])
    h = jnp.maximum(dinv * s_ref[0] + b_ref[...], 0.0)
    o_ref[0] = jnp.dot(h, w_ref[...], preferred_element_type=jnp.float32) * dinv


def _tc_pool_body(s_ref, deg_ref, b_ref, batch_ref,
                  w0_ref, b0_ref, w1_ref, b1_ref, w2_ref, b2_ref,
                  p_ref, o_ref):
    g = pl.program_id(0)
    i = pl.program_id(1)
    h = jnp.maximum(_dinv(deg_ref[0]) * s_ref[0] + b_ref[...], 0.0)
    bt = batch_ref[0, 0]
    oh = (bt[:, None] == lax.broadcasted_iota(jnp.int32, (RB, NG), 1))
    pp = lax.dot_general(oh.astype(jnp.float32), h,
                         (((0,), (0,)), ((), ())),
                         preferred_element_type=jnp.float32)

    @pl.when(jnp.logical_and(g == 0, i == 0))
    def _():
        p_ref[0] = pp

    @pl.when(jnp.logical_and(g == 0, i > 0))
    def _():
        p_ref[0] += pp

    @pl.when(jnp.logical_and(g == 1, i == 0))
    def _():
        p_ref[1] = pp

    @pl.when(jnp.logical_and(g == 1, i > 0))
    def _():
        p_ref[1] += pp

    @pl.when(jnp.logical_and(g == 1, i == NRB - 1))
    def _():
        z = (jnp.dot(p_ref[0], w0_ref[0:D, :],
                     preferred_element_type=jnp.float32)
             + jnp.dot(p_ref[1], w0_ref[D:2 * D, :],
                       preferred_element_type=jnp.float32) + b0_ref[...])
        a = jnp.maximum(z, 0.0)
        a = jnp.maximum(jnp.dot(a, w1_ref[...],
                                preferred_element_type=jnp.float32)
                        + b1_ref[...], 0.0)
        z2 = (jnp.dot(a, w2_ref[...], preferred_element_type=jnp.float32)
              + b2_ref[...])
        o_ref[...] = jax.nn.sigmoid(z2)


_tc_t0 = pl.pallas_call(
    _tc_t0_body,
    grid=(2, NRB),
    in_specs=[
        pl.BlockSpec((1, RB, D), lambda g, i: (g, i, 0)),
        pl.BlockSpec((1, RB, 1), lambda g, i: (g, i, 0)),
        pl.BlockSpec((D, D), lambda g, i: (0, 0)),
    ],
    out_specs=pl.BlockSpec((1, RB, D), lambda g, i: (g, i, 0)),
    out_shape=jax.ShapeDtypeStruct((2, N, D), jnp.float32),
)

_tc_layer = pl.pallas_call(
    _tc_layer_body,
    grid=(2, NRB),
    in_specs=[
        pl.BlockSpec((1, RB, D), lambda g, i: (g, i, 0)),
        pl.BlockSpec((1, RB, 1), lambda g, i: (g, i, 0)),
        pl.BlockSpec((1, D), lambda g, i: (0, 0)),
        pl.BlockSpec((D, D), lambda g, i: (0, 0)),
    ],
    out_specs=pl.BlockSpec((1, RB, D), lambda g, i: (g, i, 0)),
    out_shape=jax.ShapeDtypeStruct((2, N, D), jnp.float32),
)

_tc_pool = pl.pallas_call(
    _tc_pool_body,
    grid=(2, NRB),
    in_specs=[
        pl.BlockSpec((1, RB, D), lambda g, i: (g, i, 0)),
        pl.BlockSpec((1, RB, 1), lambda g, i: (g, i, 0)),
        pl.BlockSpec((1, D), lambda g, i: (0, 0)),
        pl.BlockSpec((1, 1, RB), lambda g, i: (g * NRB + i, 0, 0)),
        pl.BlockSpec((2 * D, D), lambda g, i: (0, 0)),
        pl.BlockSpec((1, D), lambda g, i: (0, 0)),
        pl.BlockSpec((D, D // 2), lambda g, i: (0, 0)),
        pl.BlockSpec((1, D // 2), lambda g, i: (0, 0)),
        pl.BlockSpec((D // 2, D), lambda g, i: (0, 0)),
        pl.BlockSpec((1, D), lambda g, i: (0, 0)),
    ],
    out_specs=[
        pl.BlockSpec((2, NG, D), lambda g, i: (0, 0, 0)),
        pl.BlockSpec((NG, D), lambda g, i: (0, 0)),
    ],
    out_shape=[
        jax.ShapeDtypeStruct((2, NG, D), jnp.float32),
        jax.ShapeDtypeStruct((NG, D), jnp.float32),
    ],
)


def kernel(x1, edge_index1, batch1, x2, edge_index2, batch2,
           Wg0, bg0, Wg1, bg1, Wg2, bg2, W0, b0, W1, b1, W2, b2):
    x_all = jnp.stack([x1, x2])
    src_cat = jnp.concatenate([edge_index1[0], edge_index2[0]]).astype(jnp.int32)
    dst_cat = jnp.concatenate([edge_index1[1], edge_index2[1]]).astype(jnp.int32)

    deg_flat, srcpad, dstpad = _sc_deg(src_cat, dst_cat)
    deg = deg_flat.reshape(2, NPAD)[:, :N].reshape(2, N, 1)
    src2d = srcpad.reshape(NT * EPTP // CSTREAM, CSTREAM)
    dst2d = dstpad.reshape(NT * EPTP // CSTREAM, CSTREAM)

    t = _tc_t0(x_all, deg, Wg0)
    for W_next, b_prev in ((Wg1, bg0), (Wg2, bg1)):
        s_ = _sc_conv(t.reshape(2 * N, D), src2d, dst2d).reshape(2, N, D)
        t = _tc_layer(s_, deg, b_prev.reshape(1, D), W_next)
    s_ = _sc_conv(t.reshape(2 * N, D), src2d, dst2d).reshape(2, N, D)

    batch3d = jnp.stack([batch1, batch2]).astype(jnp.int32).reshape(2 * NRB, 1, RB)
    W2p = jnp.pad(W2, ((0, 0), (0, D - 1)))
    b2p = jnp.pad(b2, (0, D - 1)).reshape(1, D)
    _, out = _tc_pool(s_, deg, bg2.reshape(1, D), batch3d,
                      W0, b0.reshape(1, D), W1, b1.reshape(1, D // 2),
                      W2p, b2p)
    return out[:, 0]
